# baseline (speedup 1.0000x reference)
import functools

import numpy as np

import jax
import jax.numpy as jnp
from jax import lax
from jax.experimental import pallas as pl
from jax.experimental.pallas import tpu as pltpu
from jax.experimental.pallas import tpu_sc as plsc

_E = 320000
_U = 160000
_R = 7
_D = 128

_NC = 2
_NS = 16

_CHUNK = 80
_GN = _U // _NS
_GC = _GN // _CHUNK
_MN = _E // _NS
_MC = _MN // _CHUNK
_ZB = _U // _NS
_ESL = _E // _NS


def _sc_body(el, u2d, d2u, ulen, mask, idxg_v, rowg_v, idxm_v, ones_v,
             zer_v, stage_v, el_sh, mask_sh, sem):
  c = lax.axis_index("c")
  s = lax.axis_index("s")

  @pl.when(c == 1)
  def _gather():
    pltpu.sync_copy(el.at[pl.ds(s * _ESL, _ESL)], stage_v)
    pltpu.sync_copy(stage_v, el_sh.at[pl.ds(s * _ESL, _ESL)])
    pltpu.sync_copy(u2d.at[s], idxg_v)
    plsc.subcore_barrier()
    def fire(j, carry):
      pltpu.async_copy(el_sh.at[idxg_v.at[j]],
                       rowg_v.at[pl.ds(j * _CHUNK, _CHUNK)], sem)
      return carry
    lax.fori_loop(0, _GC, fire, 0)
    def drain(j, carry):
      pltpu.make_async_copy(el_sh.at[idxg_v.at[j]],
                            rowg_v.at[pl.ds(j * _CHUNK, _CHUNK)], sem).wait()
      return carry
    lax.fori_loop(0, _GC, drain, 0)
    pltpu.sync_copy(rowg_v, ulen.at[pl.ds(s * _GN, _GN)])

  @pl.when(c == 0)
  def _mask():
    def zstore(i, carry):
      zer_v[pl.ds(i * 16, 16)] = jnp.zeros((16,), jnp.float32)
      return carry
    lax.fori_loop(0, _ZB // 16, zstore, 0)
    for k in range(_CHUNK // 16):
      ones_v[pl.ds(k * 16, 16)] = jnp.ones((16,), jnp.float32)
    pltpu.sync_copy(zer_v, mask_sh.at[pl.ds(s * _ZB, _ZB)])
    pltpu.sync_copy(d2u.at[s], idxm_v)
    plsc.subcore_barrier()
    def fire(j, carry):
      pltpu.async_copy(ones_v, mask_sh.at[idxm_v.at[j]], sem)
      return carry
    lax.fori_loop(0, _MC, fire, 0)
    def drain(j, carry):
      pltpu.make_async_copy(ones_v, mask_sh.at[idxm_v.at[j]], sem).wait()
      return carry
    lax.fori_loop(0, _MC, drain, 0)
    plsc.subcore_barrier()
    pltpu.sync_copy(mask_sh.at[pl.ds(s * _ZB, _ZB)], zer_v)
    pltpu.sync_copy(zer_v, mask.at[pl.ds(s * _ZB, _ZB)])


@functools.cache
def _make_sc_call():
  return pl.kernel(
    _sc_body,
    out_type=(
        jax.ShapeDtypeStruct((_U,), jnp.float32),
        jax.ShapeDtypeStruct((_U,), jnp.float32),
    ),
    mesh=plsc.VectorSubcoreMesh(core_axis_name="c", subcore_axis_name="s",
                                num_cores=_NC, num_subcores=_NS),
    scratch_types=[
        pltpu.VMEM((_GC, _CHUNK), jnp.int32),
        pltpu.VMEM((_GN,), jnp.float32),
        pltpu.VMEM((_MC, _CHUNK), jnp.int32),
        pltpu.VMEM((_CHUNK,), jnp.float32),
        pltpu.VMEM((_ZB,), jnp.float32),
        pltpu.VMEM((_ESL,), jnp.float32),
        pltpu.VMEM_SHARED((_E,), jnp.float32),
        pltpu.VMEM_SHARED((_U,), jnp.float32),
        pltpu.SemaphoreType.DMA,
    ],
  )


_BU = 6400
_NB = _U // _BU


def _envelope(d):
  p = 9.0
  a = -(p + 1.0) * (p + 2.0) / 2.0
  b = p * (p + 2.0)
  c = -p * (p + 1.0) / 2.0
  d2 = d * d
  d4 = d2 * d2
  d8 = d4 * d4
  env = 1.0 / d + d8 * (a + d * (b + d * c))
  return jnp.where(d < 1.0, env, 0.0)


def _tc_body(x_ref, mask_ref, w1_ref, w2_ref, wsw_ref, g_ref, b_ref,
             ef_ref, p_ref, t_ref):
  x = x_ref[...].reshape(1, _BU)
  k = lax.broadcasted_iota(jnp.int32, (_R, 1), 0)
  freqs = jnp.float32(np.pi) * (k + 1).astype(jnp.float32)
  d6 = x * (1.0 / 6.0)
  d4 = x * (1.0 / 4.0)
  n6 = (2.0 / 6.0) ** 0.5
  n4 = (2.0 / 4.0) ** 0.5
  p7 = n6 * _envelope(d6) * jnp.sin(freqs * d6)
  p_ref[...] = p7
  t_ref[...] = n4 * _envelope(d4) * jnp.sin(freqs * d4)
  m = mask_ref[...].reshape(1, _BU)
  z1 = lax.dot_general(p7, w1_ref[...], (((0,), (0,)), ((), ())),
                       preferred_element_type=jnp.float32)
  z = jnp.dot(z1, w2_ref[...], preferred_element_type=jnp.float32)
  mcol = lax.dot_general(m, jnp.ones((1, 1), jnp.float32),
                         (((0,), (0,)), ((), ())),
                         preferred_element_type=jnp.float32)
  mu = jnp.mean(z, axis=1, keepdims=True)
  zc = z - mu
  var = jnp.mean(zc * zc, axis=1, keepdims=True)
  y = (zc * lax.rsqrt(var + 1e-5) * g_ref[...] + b_ref[...]) * mcol
  sw = jnp.dot(y, wsw_ref[...], preferred_element_type=jnp.float32)
  ef_ref[...] = sw / (1.0 + jnp.exp(-sw))


def _tc_call(ulen3, mask3, w1, w2, wsw, g, b):
  return pl.pallas_call(
      _tc_body,
      grid=(_NB,),
      in_specs=[
          pl.BlockSpec((1, 1, _BU), lambda i: (i, 0, 0)),
          pl.BlockSpec((1, 1, _BU), lambda i: (i, 0, 0)),
          pl.BlockSpec((_R, _D), lambda i: (0, 0)),
          pl.BlockSpec((_D, _D), lambda i: (0, 0)),
          pl.BlockSpec((_D, _D), lambda i: (0, 0)),
          pl.BlockSpec((1, _D), lambda i: (0, 0)),
          pl.BlockSpec((1, _D), lambda i: (0, 0)),
      ],
      out_specs=[
          pl.BlockSpec((_BU, _D), lambda i: (i, 0)),
          pl.BlockSpec((_R, _BU), lambda i: (0, i)),
          pl.BlockSpec((_R, _BU), lambda i: (0, i)),
      ],
      out_shape=[
          jax.ShapeDtypeStruct((_U, _D), jnp.float32),
          jax.ShapeDtypeStruct((_R, _U), jnp.float32),
          jax.ShapeDtypeStruct((_R, _U), jnp.float32),
      ],
      compiler_params=pltpu.CompilerParams(
          dimension_semantics=("parallel",),
      ),
  )(ulen3, mask3, w1, w2, wsw, g, b)


def kernel(edge_lengths, undirected2directed, directed2undirected,
           W1, W2, Wsw, ln_gamma, ln_beta):
  u2d = undirected2directed.reshape(_NS, _GC, _CHUNK)
  d2u = directed2undirected.reshape(_NS, _MC, _CHUNK)
  ulen, mask = _make_sc_call()(edge_lengths, u2d, d2u)
  ef, prbf_t, trbf_t = _tc_call(ulen.reshape(_NB, 1, _BU),
                                mask.reshape(_NB, 1, _BU), W1, W2, Wsw,
                                ln_gamma.reshape(1, _D),
                                ln_beta.reshape(1, _D))
  return (ef, prbf_t.T, trbf_t.T)


# speedup vs baseline: 1.0409x; 1.0409x over previous
import functools

import numpy as np

import jax
import jax.numpy as jnp
from jax import lax
from jax.experimental import pallas as pl
from jax.experimental.pallas import tpu as pltpu
from jax.experimental.pallas import tpu_sc as plsc

_E = 320000
_U = 160000
_R = 7
_D = 128

_NC = 2
_NS = 16

_CHUNK = 80
_GN = _U // _NS
_GC = _GN // _CHUNK
_MN = _E // _NS
_MC = _MN // _CHUNK
_ZB = _U // _NS
_ESL = _E // _NS


def _sc_body(el, u2d, d2u, ulen, mask, idxg_v, rowg_v, idxm_v, ones_v,
             zer_v, stage_v, el_sh, mask_sh, sem):
  c = lax.axis_index("c")
  s = lax.axis_index("s")

  @pl.when(c == 1)
  def _gather():
    pltpu.sync_copy(el.at[pl.ds(s * _ESL, _ESL)], stage_v)
    pltpu.sync_copy(stage_v, el_sh.at[pl.ds(s * _ESL, _ESL)])
    pltpu.sync_copy(u2d.at[s], idxg_v)
    plsc.subcore_barrier()
    def fire(j, carry):
      pltpu.async_copy(el_sh.at[idxg_v.at[j]],
                       rowg_v.at[pl.ds(j * _CHUNK, _CHUNK)], sem)
      return carry
    lax.fori_loop(0, _GC, fire, 0)
    def drain(j, carry):
      pltpu.make_async_copy(el_sh.at[idxg_v.at[j]],
                            rowg_v.at[pl.ds(j * _CHUNK, _CHUNK)], sem).wait()
      return carry
    lax.fori_loop(0, _GC, drain, 0)
    pltpu.sync_copy(rowg_v, ulen.at[pl.ds(s * _GN, _GN)])

  @pl.when(c == 0)
  def _mask():
    def zstore(i, carry):
      zer_v[pl.ds(i * 16, 16)] = jnp.zeros((16,), jnp.float32)
      return carry
    lax.fori_loop(0, _ZB // 16, zstore, 0)
    for k in range(_CHUNK // 16):
      ones_v[pl.ds(k * 16, 16)] = jnp.ones((16,), jnp.float32)
    pltpu.sync_copy(zer_v, mask_sh.at[pl.ds(s * _ZB, _ZB)])
    pltpu.sync_copy(d2u.at[s], idxm_v)
    plsc.subcore_barrier()
    def fire(j, carry):
      pltpu.async_copy(ones_v, mask_sh.at[idxm_v.at[j]], sem)
      return carry
    lax.fori_loop(0, _MC, fire, 0)
    def drain(j, carry):
      pltpu.make_async_copy(ones_v, mask_sh.at[idxm_v.at[j]], sem).wait()
      return carry
    lax.fori_loop(0, _MC, drain, 0)
    plsc.subcore_barrier()
    pltpu.sync_copy(mask_sh.at[pl.ds(s * _ZB, _ZB)], zer_v)
    pltpu.sync_copy(zer_v, mask.at[pl.ds(s * _ZB, _ZB)])


@functools.cache
def _make_sc_call():
  return pl.kernel(
    _sc_body,
    out_type=(
        jax.ShapeDtypeStruct((_U,), jnp.float32),
        jax.ShapeDtypeStruct((_U,), jnp.float32),
    ),
    mesh=plsc.VectorSubcoreMesh(core_axis_name="c", subcore_axis_name="s",
                                num_cores=_NC, num_subcores=_NS),
    scratch_types=[
        pltpu.VMEM((_GC, _CHUNK), jnp.int32),
        pltpu.VMEM((_GN,), jnp.float32),
        pltpu.VMEM((_MC, _CHUNK), jnp.int32),
        pltpu.VMEM((_CHUNK,), jnp.float32),
        pltpu.VMEM((_ZB,), jnp.float32),
        pltpu.VMEM((_ESL,), jnp.float32),
        pltpu.VMEM_SHARED((_E,), jnp.float32),
        pltpu.VMEM_SHARED((_U,), jnp.float32),
        pltpu.SemaphoreType.DMA,
    ],
  )


_BU = 16000
_NB = _U // _BU


def _envelope(d):
  p = 9.0
  a = -(p + 1.0) * (p + 2.0) / 2.0
  b = p * (p + 2.0)
  c = -p * (p + 1.0) / 2.0
  d2 = d * d
  d4 = d2 * d2
  d8 = d4 * d4
  env = 1.0 / d + d8 * (a + d * (b + d * c))
  return jnp.where(d < 1.0, env, 0.0)


def _tc_body(x_ref, mask_ref, w1_ref, w2_ref, wsw_ref, g_ref, b_ref,
             ef_ref, p_ref, t_ref):
  x = x_ref[...].reshape(1, _BU)
  k = lax.broadcasted_iota(jnp.int32, (_R, 1), 0)
  freqs = jnp.float32(np.pi) * (k + 1).astype(jnp.float32)
  d6 = x * (1.0 / 6.0)
  d4 = x * (1.0 / 4.0)
  n6 = (2.0 / 6.0) ** 0.5
  n4 = (2.0 / 4.0) ** 0.5
  p7 = n6 * _envelope(d6) * jnp.sin(freqs * d6)
  p_ref[...] = p7
  t_ref[...] = n4 * _envelope(d4) * jnp.sin(freqs * d4)
  m = mask_ref[...].reshape(1, _BU)
  z1 = lax.dot_general(p7, w1_ref[...], (((0,), (0,)), ((), ())),
                       preferred_element_type=jnp.float32)
  z = jnp.dot(z1, w2_ref[...], preferred_element_type=jnp.float32)
  mcol = lax.transpose(m, (1, 0))
  mu = jnp.mean(z, axis=1, keepdims=True)
  zc = z - mu
  var = jnp.mean(zc * zc, axis=1, keepdims=True)
  y = (zc * lax.rsqrt(var + 1e-5) * g_ref[...] + b_ref[...]) * mcol
  sw = jnp.dot(y, wsw_ref[...], preferred_element_type=jnp.float32)
  ef_ref[...] = sw / (1.0 + jnp.exp(-sw))


def _tc_call(ulen3, mask3, w1, w2, wsw, g, b):
  return pl.pallas_call(
      _tc_body,
      grid=(_NB,),
      in_specs=[
          pl.BlockSpec((1, 1, _BU), lambda i: (i, 0, 0)),
          pl.BlockSpec((1, 1, _BU), lambda i: (i, 0, 0)),
          pl.BlockSpec((_R, _D), lambda i: (0, 0)),
          pl.BlockSpec((_D, _D), lambda i: (0, 0)),
          pl.BlockSpec((_D, _D), lambda i: (0, 0)),
          pl.BlockSpec((1, _D), lambda i: (0, 0)),
          pl.BlockSpec((1, _D), lambda i: (0, 0)),
      ],
      out_specs=[
          pl.BlockSpec((_BU, _D), lambda i: (i, 0)),
          pl.BlockSpec((_R, _BU), lambda i: (0, i)),
          pl.BlockSpec((_R, _BU), lambda i: (0, i)),
      ],
      out_shape=[
          jax.ShapeDtypeStruct((_U, _D), jnp.float32),
          jax.ShapeDtypeStruct((_R, _U), jnp.float32),
          jax.ShapeDtypeStruct((_R, _U), jnp.float32),
      ],
      compiler_params=pltpu.CompilerParams(
          dimension_semantics=("parallel",),
      ),
  )(ulen3, mask3, w1, w2, wsw, g, b)


def kernel(edge_lengths, undirected2directed, directed2undirected,
           W1, W2, Wsw, ln_gamma, ln_beta):
  u2d = undirected2directed.reshape(_NS, _GC, _CHUNK)
  d2u = directed2undirected.reshape(_NS, _MC, _CHUNK)
  ulen, mask = _make_sc_call()(edge_lengths, u2d, d2u)
  ef, prbf_t, trbf_t = _tc_call(ulen.reshape(_NB, 1, _BU),
                                mask.reshape(_NB, 1, _BU), W1, W2, Wsw,
                                ln_gamma.reshape(1, _D),
                                ln_beta.reshape(1, _D))
  return (ef, prbf_t.T, trbf_t.T)


# speedup vs baseline: 1.0645x; 1.0227x over previous
import functools

import numpy as np

import jax
import jax.numpy as jnp
from jax import lax
from jax.experimental import pallas as pl
from jax.experimental.pallas import tpu as pltpu
from jax.experimental.pallas import tpu_sc as plsc

_E = 320000
_U = 160000
_R = 7
_D = 128

_NC = 2
_NS = 16

_CHUNK = 80
_GN = _U // _NS
_GC = _GN // _CHUNK
_MN = _E // _NS
_MC = _MN // _CHUNK
_ZB = _U // _NS
_ESL = _E // _NS


def _sc_body(el, u2d, d2u, ulen, mask, idxg_v, rowg_v, idxm_v, ones_v,
             zer_v, stage_v, el_sh, mask_sh, sem):
  c = lax.axis_index("c")
  s = lax.axis_index("s")

  @pl.when(c == 1)
  def _gather():
    pltpu.sync_copy(el.at[pl.ds(s * _ESL, _ESL)], stage_v)
    pltpu.sync_copy(stage_v, el_sh.at[pl.ds(s * _ESL, _ESL)])
    pltpu.sync_copy(u2d.at[pl.ds(s * _GN, _GN)], idxg_v)
    plsc.subcore_barrier()
    def fire(j, carry):
      pltpu.async_copy(el_sh.at[idxg_v.at[pl.ds(j * _CHUNK, _CHUNK)]],
                       rowg_v.at[pl.ds(j * _CHUNK, _CHUNK)], sem)
      return carry
    lax.fori_loop(0, _GC, fire, 0)
    def drain(j, carry):
      pltpu.make_async_copy(el_sh.at[idxg_v.at[pl.ds(j * _CHUNK, _CHUNK)]],
                            rowg_v.at[pl.ds(j * _CHUNK, _CHUNK)], sem).wait()
      return carry
    lax.fori_loop(0, _GC, drain, 0)
    pltpu.sync_copy(rowg_v, ulen.at[pl.ds(s * _GN, _GN)])

  @pl.when(c == 0)
  def _mask():
    def zstore(i, carry):
      zer_v[pl.ds(i * 16, 16)] = jnp.zeros((16,), jnp.float32)
      return carry
    lax.fori_loop(0, _ZB // 16, zstore, 0)
    for k in range(_CHUNK // 16):
      ones_v[pl.ds(k * 16, 16)] = jnp.ones((16,), jnp.float32)
    pltpu.sync_copy(zer_v, mask_sh.at[pl.ds(s * _ZB, _ZB)])
    pltpu.sync_copy(d2u.at[pl.ds(s * _MN, _MN)], idxm_v)
    plsc.subcore_barrier()
    def fire(j, carry):
      pltpu.async_copy(ones_v,
                       mask_sh.at[idxm_v.at[pl.ds(j * _CHUNK, _CHUNK)]], sem)
      return carry
    lax.fori_loop(0, _MC, fire, 0)
    def drain(j, carry):
      pltpu.make_async_copy(
          ones_v, mask_sh.at[idxm_v.at[pl.ds(j * _CHUNK, _CHUNK)]],
          sem).wait()
      return carry
    lax.fori_loop(0, _MC, drain, 0)
    plsc.subcore_barrier()
    pltpu.sync_copy(mask_sh.at[pl.ds(s * _ZB, _ZB)], zer_v)
    pltpu.sync_copy(zer_v, mask.at[pl.ds(s * _ZB, _ZB)])


@functools.cache
def _make_sc_call():
  return pl.kernel(
    _sc_body,
    out_type=(
        jax.ShapeDtypeStruct((_U,), jnp.float32),
        jax.ShapeDtypeStruct((_U,), jnp.float32),
    ),
    mesh=plsc.VectorSubcoreMesh(core_axis_name="c", subcore_axis_name="s",
                                num_cores=_NC, num_subcores=_NS),
    scratch_types=[
        pltpu.VMEM((_GN,), jnp.int32),
        pltpu.VMEM((_GN,), jnp.float32),
        pltpu.VMEM((_MN,), jnp.int32),
        pltpu.VMEM((_CHUNK,), jnp.float32),
        pltpu.VMEM((_ZB,), jnp.float32),
        pltpu.VMEM((_ESL,), jnp.float32),
        pltpu.VMEM_SHARED((_E,), jnp.float32),
        pltpu.VMEM_SHARED((_U,), jnp.float32),
        pltpu.SemaphoreType.DMA,
    ],
  )


_BU = 16000
_NB = _U // _BU


def _envelope(d):
  p = 9.0
  a = -(p + 1.0) * (p + 2.0) / 2.0
  b = p * (p + 2.0)
  c = -p * (p + 1.0) / 2.0
  d2 = d * d
  d4 = d2 * d2
  d8 = d4 * d4
  env = 1.0 / d + d8 * (a + d * (b + d * c))
  return jnp.where(d < 1.0, env, 0.0)


def _tc_body(x_ref, mask_ref, w1_ref, w2_ref, wsw_ref, g_ref, b_ref,
             ef_ref, p_ref, t_ref):
  x = x_ref[...].reshape(1, _BU)
  k = lax.broadcasted_iota(jnp.int32, (_R, 1), 0)
  freqs = jnp.float32(np.pi) * (k + 1).astype(jnp.float32)
  d6 = x * (1.0 / 6.0)
  d4 = x * (1.0 / 4.0)
  n6 = (2.0 / 6.0) ** 0.5
  n4 = (2.0 / 4.0) ** 0.5
  p7 = n6 * _envelope(d6) * jnp.sin(freqs * d6)
  p_ref[...] = p7
  t_ref[...] = n4 * _envelope(d4) * jnp.sin(freqs * d4)
  m = mask_ref[...].reshape(1, _BU)
  z1 = lax.dot_general(p7, w1_ref[...], (((0,), (0,)), ((), ())),
                       preferred_element_type=jnp.float32)
  z = jnp.dot(z1, w2_ref[...], preferred_element_type=jnp.float32)
  mcol = lax.transpose(m, (1, 0))
  mu = jnp.mean(z, axis=1, keepdims=True)
  zc = z - mu
  var = jnp.mean(zc * zc, axis=1, keepdims=True)
  y = (zc * lax.rsqrt(var + 1e-5) * g_ref[...] + b_ref[...]) * mcol
  sw = jnp.dot(y, wsw_ref[...], preferred_element_type=jnp.float32)
  ef_ref[...] = sw / (1.0 + jnp.exp(-sw))


def _tc_call(ulen3, mask3, w1, w2, wsw, g, b):
  return pl.pallas_call(
      _tc_body,
      grid=(_NB,),
      in_specs=[
          pl.BlockSpec((1, 1, _BU), lambda i: (i, 0, 0)),
          pl.BlockSpec((1, 1, _BU), lambda i: (i, 0, 0)),
          pl.BlockSpec((_R, _D), lambda i: (0, 0)),
          pl.BlockSpec((_D, _D), lambda i: (0, 0)),
          pl.BlockSpec((_D, _D), lambda i: (0, 0)),
          pl.BlockSpec((1, _D), lambda i: (0, 0)),
          pl.BlockSpec((1, _D), lambda i: (0, 0)),
      ],
      out_specs=[
          pl.BlockSpec((_BU, _D), lambda i: (i, 0)),
          pl.BlockSpec((_R, _BU), lambda i: (0, i)),
          pl.BlockSpec((_R, _BU), lambda i: (0, i)),
      ],
      out_shape=[
          jax.ShapeDtypeStruct((_U, _D), jnp.float32),
          jax.ShapeDtypeStruct((_R, _U), jnp.float32),
          jax.ShapeDtypeStruct((_R, _U), jnp.float32),
      ],
      compiler_params=pltpu.CompilerParams(
          dimension_semantics=("parallel",),
      ),
  )(ulen3, mask3, w1, w2, wsw, g, b)


def kernel(edge_lengths, undirected2directed, directed2undirected,
           W1, W2, Wsw, ln_gamma, ln_beta):
  ulen, mask = _make_sc_call()(edge_lengths, undirected2directed,
                               directed2undirected)
  ef, prbf_t, trbf_t = _tc_call(ulen.reshape(_NB, 1, _BU),
                                mask.reshape(_NB, 1, _BU), W1, W2, Wsw,
                                ln_gamma.reshape(1, _D),
                                ln_beta.reshape(1, _D))
  return (ef, prbf_t.T, trbf_t.T)
